# baseline (speedup 1.0000x reference)
import functools

import jax
import jax.numpy as jnp
from jax import lax
from jax.experimental import pallas as pl
from jax.experimental.pallas import tpu as pltpu
from jax.experimental.pallas import tpu_sc as plsc

_NUM_FIELDS = 26
_VOCAB = 100000
_EMB = 32
_BATCH = 1024
_SEQ = 50
_NUM_DIM = 16
_LANES = 16

_FS = _NUM_FIELDS * _SEQ
_CHUNK = 112
_NCHUNK = 12
_FS_PAD = _NCHUNK * _CHUNK
_OUT_ROWS = _NUM_DIM + _NUM_FIELDS * _EMB
_OUT_FLAT = _OUT_ROWS * _SEQ
_NUM_FLAT = _NUM_DIM * _SEQ


def _build_sc_call():
    info = plsc.get_sparse_core_info()
    nc, ns = info.num_cores, info.num_subcores
    nw = nc * ns
    bpw = _BATCH // nw

    mesh = plsc.VectorSubcoreMesh(core_axis_name="c", subcore_axis_name="s")

    @functools.partial(
        pl.kernel,
        mesh=mesh,
        compiler_params=pltpu.CompilerParams(
            needs_layout_passes=False, use_tc_tiling_on_sc=False),
        out_type=jax.ShapeDtypeStruct((_BATCH, _OUT_FLAT), jnp.float32),
        scratch_types=[
            pltpu.VMEM((_FS_PAD,), jnp.int32),
            pltpu.VMEM((_NCHUNK, _CHUNK), jnp.int32),
            pltpu.VMEM((_NCHUNK, _CHUNK), jnp.int32),
            pltpu.VMEM((_FS_PAD, _EMB), jnp.float32),
            pltpu.VMEM((_OUT_FLAT,), jnp.float32),
            pltpu.SemaphoreType.DMA,
        ],
    )
    def fn(tab, cat, num, offs, out, catv, offsv, gidx, gbuf, obuf, gsem):
        wid = lax.axis_index("s") * nc + lax.axis_index("c")
        stride = lax.iota(jnp.int32, _LANES) * _SEQ
        pltpu.sync_copy(offs, offsv)

        def batch_body(t, carry):
            b = wid * bpw + t
            pltpu.sync_copy(cat.at[b], catv)
            for j in range(_NCHUNK):
                for q in range(_CHUNK // _LANES):
                    sl = pl.ds(q * _LANES, _LANES)
                    gidx[j, sl] = catv[pl.ds(j * _CHUNK + q * _LANES, _LANES)] + offsv[j, sl]
            copies = [
                pltpu.async_copy(tab.at[gidx.at[j]],
                                 gbuf.at[pl.ds(j * _CHUNK, _CHUNK)], gsem)
                for j in range(_NCHUNK)
            ]
            pltpu.sync_copy(num.at[b], obuf.at[pl.ds(0, _NUM_FLAT)])
            for c in copies:
                c.wait()

            def field_body(d, c2):
                rbase = d * _SEQ
                obase = _NUM_FLAT + d * (_EMB * _SEQ)
                for s in range(_SEQ):
                    v0 = gbuf[rbase + s, pl.ds(0, _LANES)]
                    v1 = gbuf[rbase + s, pl.ds(_LANES, _LANES)]
                    i0 = (obase + s) + stride
                    plsc.store_scatter(obuf, [i0], v0)
                    plsc.store_scatter(obuf, [i0 + _LANES * _SEQ], v1)
                return c2

            lax.fori_loop(0, _NUM_FIELDS, field_body, 0)
            pltpu.sync_copy(obuf, out.at[b])
            return carry

        lax.fori_loop(0, bpw, batch_body, 0)

    return fn


def kernel(num, cat, tables):
    tab = tables.reshape(_NUM_FIELDS * _VOCAB, _EMB)
    cat2 = cat.astype(jnp.int32).reshape(_BATCH, _FS)
    cat3 = jnp.pad(cat2, ((0, 0), (0, _FS_PAD - _FS)))
    num2 = num.reshape(_BATCH, _NUM_FLAT)
    pos = jnp.arange(_FS_PAD, dtype=jnp.int32)
    offs = jnp.where(pos < _FS, (pos // _SEQ) * _VOCAB, 0).reshape(_NCHUNK, _CHUNK)
    out = _build_sc_call()(tab, cat3, num2, offs)
    return out.reshape(_BATCH, _OUT_ROWS, _SEQ)


# speedup vs baseline: 5.9428x; 5.9428x over previous
import functools

import jax
import jax.numpy as jnp
from jax import lax
from jax.experimental import pallas as pl
from jax.experimental.pallas import tpu as pltpu
from jax.experimental.pallas import tpu_sc as plsc

_NUM_FIELDS = 26
_VOCAB = 100000
_EMB = 32
_BATCH = 1024
_SEQ = 50
_NUM_DIM = 16
_LANES = 16

_OUT_ROWS = _NUM_DIM + _NUM_FIELDS * _EMB
_SB = 8
_QB = _BATCH // _LANES
_SBLOCKS = ((0, 8), (8, 8), (16, 8), (24, 8), (32, 8), (40, 8), (48, 2))


def _build_sc_call():
    info = plsc.get_sparse_core_info()
    nc, ns = info.num_cores, info.num_subcores
    nw = nc * ns
    ppw = (_NUM_FIELDS * _EMB) // nw

    mesh = plsc.VectorSubcoreMesh(core_axis_name="c", subcore_axis_name="s")

    @functools.partial(
        pl.kernel,
        mesh=mesh,
        compiler_params=pltpu.CompilerParams(needs_layout_passes=False),
        out_type=jax.ShapeDtypeStruct((_SEQ, _OUT_ROWS, _BATCH), jnp.float32),
        scratch_types=[
            pltpu.VMEM((_VOCAB,), jnp.float32),
            pltpu.VMEM((_SB, _BATCH), jnp.int32),
            pltpu.VMEM((_SB, _BATCH), jnp.float32),
            pltpu.VMEM((8, _BATCH), jnp.float32),
            pltpu.SemaphoreType.DMA,
        ],
    )
    def fn(tab, cat, num, out, rowb, catb, outb, numb, rsem):
        wid = lax.axis_index("s") * nc + lax.axis_index("c")

        def num_slab(m):
            s = m // 2
            k0 = (m % 2) * 8
            pltpu.sync_copy(num.at[s, pl.ds(k0, 8)], numb)
            pltpu.sync_copy(numb, out.at[s, pl.ds(k0, 8)])

        num_slab(wid)
        num_slab(wid + 32)
        num_slab(wid + 64)

        @pl.when(wid < 4)
        def _():
            num_slab(wid + 96)

        def unit_body(k, carry):
            p = wid * ppw + k
            d = p // _EMB
            e = p - d * _EMB
            c = _NUM_DIM + _EMB * d + e
            pltpu.async_copy(tab.at[d, e], rowb, rsem).wait()
            for s0, rows in _SBLOCKS:
                pltpu.sync_copy(cat.at[d, pl.ds(s0, rows)],
                                catb.at[pl.ds(0, rows)])
                for si in range(rows):
                    @plsc.parallel_loop(0, _QB, unroll=8)
                    def _(q, si=si):
                        sl = pl.ds(q * _LANES, _LANES)
                        idx = catb[si, sl]
                        outb[si, sl] = plsc.load_gather(rowb, [idx])
                pltpu.sync_copy(outb.at[pl.ds(0, rows)],
                                out.at[pl.ds(s0, rows), c])
            return carry

        lax.fori_loop(0, ppw, unit_body, 0)

    return fn


def kernel(num, cat, tables):
    tab_t = jnp.transpose(tables, (0, 2, 1))
    cat_t = jnp.transpose(cat.astype(jnp.int32), (1, 2, 0))
    num_t = jnp.transpose(num, (2, 1, 0))
    out_t = _build_sc_call()(tab_t, cat_t, num_t)
    return jnp.transpose(out_t, (2, 1, 0))


# speedup vs baseline: 8.2587x; 1.3897x over previous
import functools

import jax
import jax.numpy as jnp
from jax import lax
from jax.experimental import pallas as pl
from jax.experimental.pallas import tpu as pltpu
from jax.experimental.pallas import tpu_sc as plsc

_NUM_FIELDS = 26
_VOCAB = 100000
_EMB = 32
_BATCH = 1024
_SEQ = 50
_NUM_DIM = 16
_LANES = 16

_OUT_ROWS = _NUM_DIM + _NUM_FIELDS * _EMB
_SB = 4
_NBLK = 13
_QB = _BATCH // _LANES


def _build_sc_call():
    info = plsc.get_sparse_core_info()
    nc, ns = info.num_cores, info.num_subcores
    nw = nc * ns
    ppw = (_NUM_FIELDS * _EMB) // nw

    mesh = plsc.VectorSubcoreMesh(core_axis_name="c", subcore_axis_name="s")

    @functools.partial(
        pl.kernel,
        mesh=mesh,
        compiler_params=pltpu.CompilerParams(needs_layout_passes=False),
        out_type=jax.ShapeDtypeStruct((_SEQ, _OUT_ROWS, _BATCH), jnp.float32),
        scratch_types=[
            pltpu.VMEM((_VOCAB,), jnp.float32),
            pltpu.VMEM((_SB, _BATCH), jnp.int32),
            pltpu.VMEM((_SB, _BATCH), jnp.int32),
            pltpu.VMEM((_SB, _BATCH), jnp.float32),
            pltpu.VMEM((_SB, _BATCH), jnp.float32),
            pltpu.SemaphoreType.DMA,
            pltpu.SemaphoreType.DMA,
            pltpu.SemaphoreType.DMA,
            pltpu.SemaphoreType.DMA,
            pltpu.SemaphoreType.DMA,
        ],
    )
    def fn(tab, cat, num, out, rowb, cb0, cb1, ob0, ob1,
           rsem, cs0, cs1, os0, os1):
        wid = lax.axis_index("s") * nc + lax.axis_index("c")
        dyn0 = wid * 0
        catb = (cb0, cb1)
        outb = (ob0, ob1)
        csem = (cs0, cs1)
        osem = (os0, os1)

        def num_slab(m):
            s = m // 4
            k0 = (m % 4) * 4
            pltpu.sync_copy(num.at[s, pl.ds(k0, _SB)], ob0)
            pltpu.sync_copy(ob0, out.at[s, pl.ds(k0, _SB)])

        for t in range(6):
            num_slab(wid + 32 * t)

        @pl.when(wid < 8)
        def _():
            num_slab(wid + 192)

        def unit_body(k, carry):
            p = wid * ppw + k
            d = p // _EMB
            e = p - d * _EMB
            c = _NUM_DIM + _EMB * d + e
            hrow = pltpu.async_copy(tab.at[d, e], rowb, rsem)
            hcat = {0: pltpu.async_copy(
                cat.at[d, pl.ds(dyn0, _SB)], catb[0], csem[0])}
            hout = {}
            hrow.wait()
            for j in range(_NBLK):
                b = j % 2
                rows = _SB if j < _NBLK - 1 else _SEQ - _SB * (_NBLK - 1)
                if j + 1 < _NBLK:
                    nrows = (_SB if j + 1 < _NBLK - 1
                             else _SEQ - _SB * (_NBLK - 1))
                    hcat[j + 1] = pltpu.async_copy(
                        cat.at[d, pl.ds(dyn0 + _SB * (j + 1), nrows)],
                        catb[1 - b].at[pl.ds(0, nrows)], csem[1 - b])
                hcat.pop(j).wait()
                if j - 2 in hout:
                    hout.pop(j - 2).wait()
                for si in range(rows):
                    @plsc.parallel_loop(0, _QB, unroll=8)
                    def _(q, si=si, b=b):
                        sl = pl.ds(q * _LANES, _LANES)
                        idx = catb[b][si, sl]
                        outb[b][si, sl] = plsc.load_gather(rowb, [idx])
                hout[j] = pltpu.async_copy(
                    outb[b].at[pl.ds(0, rows)],
                    out.at[pl.ds(dyn0 + _SB * j, rows), c], osem[b])
            hout.pop(_NBLK - 2).wait()
            hout.pop(_NBLK - 1).wait()
            return carry

        lax.fori_loop(0, ppw, unit_body, 0)

    return fn


def kernel(num, cat, tables):
    tab_t = jnp.transpose(tables, (0, 2, 1))
    cat_t = jnp.transpose(cat.astype(jnp.int32), (1, 2, 0))
    num_t = jnp.transpose(num, (2, 1, 0))
    out_t = _build_sc_call()(tab_t, cat_t, num_t)
    return jnp.transpose(out_t, (2, 1, 0))
